# baseline (speedup 1.0000x reference)
import functools

import numpy as np
import jax
import jax.numpy as jnp
from jax.experimental import pallas as pl
from jax.experimental.pallas import tpu as pltpu

EPS = 1e-8
TB = 512


def _bf16_round(v):
    return v.astype(jnp.bfloat16).astype(jnp.float32)


def _mm(lhs, rhs):
    return jax.lax.dot_general(
        lhs, rhs, dimension_numbers=(((1,), (0,)), ((), ())),
        preferred_element_type=jnp.float32)


_TWO_OVER_PI = 0.63661977236758134
_PIO2_HI = 1.57079625129699707031
_PIO2_MID = 7.54978941586159635335e-08
_PIO2_LO = 5.39030252995776476554e-15
_S1, _S2, _S3 = -1.66666546e-01, 8.33216087e-03, -1.95152959e-04
_C1, _C2, _C3, _C4 = (-4.99999997e-01, 4.16666233e-02, -1.38867637e-03,
                      2.43904487e-05)


def _sincos(x):
    kf = jnp.round(x * _TWO_OVER_PI)
    ki = kf.astype(jnp.int32)
    r = x - kf * _PIO2_HI
    r = r - kf * _PIO2_MID
    r = r - kf * _PIO2_LO
    r2 = r * r
    sin_r = r + r * r2 * (_S1 + r2 * (_S2 + r2 * _S3))
    cos_r = 1.0 + r2 * (_C1 + r2 * (_C2 + r2 * (_C3 + r2 * _C4)))
    swap = (ki & 1) != 0
    s_base = jnp.where(swap, cos_r, sin_r)
    c_base = jnp.where(swap, sin_r, cos_r)
    s_sign = jnp.where((ki & 2) != 0, -1.0, 1.0)
    c_sign = jnp.where(((ki + 1) & 2) != 0, -1.0, 1.0)
    return s_base * s_sign, c_base * c_sign


def _fused_kernel(q_hbm, sz_ref, wq_ref, bq_ref, rm_ref, wout_ref,
                  bout_ref, out_hbm, q_buf, in_sems, out_buf, out_sems,
                  mm_scratch, attn_scratch, *, H, D, P, d_recip, nt, nb):
    b = pl.program_id(0)
    t = pl.program_id(1)
    i = b * nt + t
    slot = jax.lax.rem(i, 2)
    nslot = jax.lax.rem(i + 1, 2)
    t_next = jax.lax.rem(t + 1, nt)
    b_next = b + jnp.where(t + 1 == nt, 1, 0)
    b_next = jnp.minimum(b_next, nb - 1)

    def in_cp(bb, tt, sl):
        return pltpu.make_async_copy(
            q_hbm.at[pl.ds(tt * TB, TB), bb, :],
            q_buf.at[sl], in_sems.at[sl])

    @pl.when(i == 0)
    def _():
        in_cp(b, t, slot).start()

    @pl.when(i + 1 < nb * nt)
    def _():
        in_cp(b_next, t_next, nslot).start()

    in_cp(b, t, slot).wait()

    xq = q_buf[slot].astype(jnp.bfloat16)
    mm_scratch[...] = _mm(xq, wq_ref[...])
    x = (mm_scratch[...] + bq_ref[...]) * d_recip
    xb = x.astype(jnp.bfloat16)
    wx = _mm(xb, rm_ref[...])
    sin_wx, cos_wx = _sincos(wx)
    for h in range(H):
        phi_h = (jnp.concatenate(
            [sin_wx[:, h * P:(h + 1) * P], cos_wx[:, h * P:(h + 1) * P]],
            axis=1) * 0.125).astype(jnp.bfloat16)
        qsz = _mm(phi_h, sz_ref[h])
        qz = jnp.maximum(qsz[:, D:D + 1], EPS)
        attn_scratch[:, h * D:(h + 1) * D] = (
            (qsz[:, :D] / qz).astype(jnp.bfloat16))
    def out_cp(tt, sl):
        return pltpu.make_async_copy(
            out_buf.at[sl], out_hbm.at[pl.ds(tt * TB, TB), b, :],
            out_sems.at[sl])

    @pl.when(i >= 2)
    def _():
        out_cp(t, slot).wait()

    mm_scratch[...] = _mm(attn_scratch[...], wout_ref[...])
    out_buf[slot] = mm_scratch[...] + bout_ref[...]
    out_cp(t, slot).start()

    @pl.when(i == nb * nt - 1)
    def _():
        out_cp(t, slot).wait()

    @pl.when((i == nb * nt - 1) & (nb * nt > 1))
    def _():
        out_cp(t, nslot).wait()


def kernel(query, s, z, random_matrices, W_q, b_q, W_out, b_out):
    T, B, E = query.shape
    _, H, twoP, D = s.shape
    P = twoP // 2
    nt = T // TB

    d_recip = float(np.float32(1.0) / np.float32(float(D) ** 0.25))

    wq_t = W_q.T
    bq = b_q.reshape(1, E)
    wout_t = W_out.T
    bout = b_out.reshape(1, E)

    rm_t = jnp.transpose(random_matrices, (0, 2, 1))
    rm_bd = jax.scipy.linalg.block_diag(*[rm_t[h] for h in range(H)])

    z_col = z[..., None]
    pad = jnp.zeros((B, H, twoP, D - 1), jnp.float32)
    sz = jnp.concatenate([s, z_col, pad], axis=-1)

    grid = (B, nt)
    out = pl.pallas_call(
        functools.partial(_fused_kernel, H=H, D=D, P=P, d_recip=d_recip,
                          nt=nt, nb=B),
        grid=grid,
        in_specs=[
            pl.BlockSpec(memory_space=pl.ANY),
            pl.BlockSpec((None, H, twoP, 2 * D), lambda b, t: (b, 0, 0, 0)),
            pl.BlockSpec((E, E), lambda b, t: (0, 0)),
            pl.BlockSpec((1, E), lambda b, t: (0, 0)),
            pl.BlockSpec((E, H * P), lambda b, t: (0, 0)),
            pl.BlockSpec((E, E), lambda b, t: (0, 0)),
            pl.BlockSpec((1, E), lambda b, t: (0, 0)),
        ],
        out_specs=pl.BlockSpec(memory_space=pl.ANY),
        out_shape=jax.ShapeDtypeStruct((T, B, E), jnp.float32),
        scratch_shapes=[
            pltpu.VMEM((2, TB, E), jnp.float32),
            pltpu.SemaphoreType.DMA((2,)),
            pltpu.VMEM((2, TB, E), jnp.float32),
            pltpu.SemaphoreType.DMA((2,)),
            pltpu.VMEM((TB, E), jnp.float32),
            pltpu.VMEM((TB, E), jnp.bfloat16),
        ],
        compiler_params=pltpu.CompilerParams(
            dimension_semantics=("arbitrary", "arbitrary"),
            vmem_limit_bytes=100 * 1024 * 1024,
        ),
    )(query, sz, wq_t, bq, rm_bd, wout_t, bout)
    return out


# speedup vs baseline: 1.0541x; 1.0541x over previous
import functools

import numpy as np
import jax
import jax.numpy as jnp
from jax.experimental import pallas as pl
from jax.experimental.pallas import tpu as pltpu

EPS = 1e-8
TB = 1024


def _bf16_round(v):
    return v.astype(jnp.bfloat16).astype(jnp.float32)


def _mm(lhs, rhs):
    return jax.lax.dot_general(
        lhs, rhs, dimension_numbers=(((1,), (0,)), ((), ())),
        preferred_element_type=jnp.float32)


_TWO_OVER_PI = 0.63661977236758134
_PIO2_HI = 1.57079625129699707031
_PIO2_MID = 7.54978941586159635335e-08
_PIO2_LO = 5.39030252995776476554e-15
_S1, _S2, _S3 = -1.66666546e-01, 8.33216087e-03, -1.95152959e-04
_C1, _C2, _C3, _C4 = (-4.99999997e-01, 4.16666233e-02, -1.38867637e-03,
                      2.43904487e-05)


def _sincos(x):
    ki = jnp.round(x * _TWO_OVER_PI).astype(jnp.int32)
    kf = ki.astype(jnp.float32)
    r = x - kf * _PIO2_HI
    r = r - kf * _PIO2_MID
    r2 = r * r
    sin_r = r + r * r2 * (_S1 + r2 * (_S2 + r2 * _S3))
    cos_r = 1.0 + r2 * (_C1 + r2 * (_C2 + r2 * (_C3 + r2 * _C4)))
    swap = (ki & 1) != 0
    s_base = jnp.where(swap, cos_r, sin_r)
    c_base = jnp.where(swap, sin_r, cos_r)
    s_bits = jax.lax.shift_left((ki & 2), 30)
    c_bits = jax.lax.shift_left(((ki + 1) & 2), 30)
    s_out = jax.lax.bitcast_convert_type(
        jax.lax.bitcast_convert_type(s_base, jnp.int32) ^ s_bits, jnp.float32)
    c_out = jax.lax.bitcast_convert_type(
        jax.lax.bitcast_convert_type(c_base, jnp.int32) ^ c_bits, jnp.float32)
    return s_out, c_out


def _fused_kernel(q_hbm, sz_ref, wq_ref, bq_ref, rm_ref, wout_ref,
                  bout_ref, out_hbm, q_buf, in_sems, out_buf, out_sems,
                  mm_scratch, attn_scratch, *, H, D, P, d_recip, nt, nb):
    b = pl.program_id(0)
    t = pl.program_id(1)
    i = b * nt + t
    slot = jax.lax.rem(i, 2)
    nslot = jax.lax.rem(i + 1, 2)
    t_next = jax.lax.rem(t + 1, nt)
    b_next = b + jnp.where(t + 1 == nt, 1, 0)
    b_next = jnp.minimum(b_next, nb - 1)

    def in_cp(bb, tt, sl):
        return pltpu.make_async_copy(
            q_hbm.at[pl.ds(tt * TB, TB), bb, :],
            q_buf.at[sl], in_sems.at[sl])

    @pl.when(i == 0)
    def _():
        in_cp(b, t, slot).start()

    @pl.when(i + 1 < nb * nt)
    def _():
        in_cp(b_next, t_next, nslot).start()

    in_cp(b, t, slot).wait()

    xq = q_buf[slot].astype(jnp.bfloat16)
    mm_scratch[...] = _mm(xq, wq_ref[...])
    x = (mm_scratch[...] + bq_ref[...]) * d_recip
    xb = x.astype(jnp.bfloat16)
    wx = _mm(xb, rm_ref[...])
    sin_wx, cos_wx = _sincos(wx)
    for h in range(H):
        phi_h = (jnp.concatenate(
            [sin_wx[:, h * P:(h + 1) * P], cos_wx[:, h * P:(h + 1) * P]],
            axis=1) * 0.125).astype(jnp.bfloat16)
        qsz = _mm(phi_h, sz_ref[h])
        qz = jnp.maximum(qsz[:, D:D + 1], EPS)
        attn_scratch[:, h * D:(h + 1) * D] = (
            (qsz[:, :D] / qz).astype(jnp.bfloat16))
    def out_cp(tt, sl):
        return pltpu.make_async_copy(
            out_buf.at[sl], out_hbm.at[pl.ds(tt * TB, TB), b, :],
            out_sems.at[sl])

    @pl.when(i >= 2)
    def _():
        out_cp(t, slot).wait()

    mm_scratch[...] = _mm(attn_scratch[...], wout_ref[...])
    out_buf[slot] = mm_scratch[...] + bout_ref[...]
    out_cp(t, slot).start()

    @pl.when(i == nb * nt - 1)
    def _():
        out_cp(t, slot).wait()

    @pl.when((i == nb * nt - 1) & (nb * nt > 1))
    def _():
        out_cp(t, nslot).wait()


def kernel(query, s, z, random_matrices, W_q, b_q, W_out, b_out):
    T, B, E = query.shape
    _, H, twoP, D = s.shape
    P = twoP // 2
    nt = T // TB

    d_recip = float(np.float32(1.0) / np.float32(float(D) ** 0.25))

    wq_t = W_q.T
    bq = b_q.reshape(1, E)
    wout_t = W_out.T
    bout = b_out.reshape(1, E)

    rm_t = jnp.transpose(random_matrices, (0, 2, 1))
    rm_bd = jax.scipy.linalg.block_diag(*[rm_t[h] for h in range(H)])

    z_col = z[..., None]
    pad = jnp.zeros((B, H, twoP, D - 1), jnp.float32)
    sz = jnp.concatenate([s, z_col, pad], axis=-1)

    grid = (B, nt)
    out = pl.pallas_call(
        functools.partial(_fused_kernel, H=H, D=D, P=P, d_recip=d_recip,
                          nt=nt, nb=B),
        grid=grid,
        in_specs=[
            pl.BlockSpec(memory_space=pl.ANY),
            pl.BlockSpec((None, H, twoP, 2 * D), lambda b, t: (b, 0, 0, 0)),
            pl.BlockSpec((E, E), lambda b, t: (0, 0)),
            pl.BlockSpec((1, E), lambda b, t: (0, 0)),
            pl.BlockSpec((E, H * P), lambda b, t: (0, 0)),
            pl.BlockSpec((E, E), lambda b, t: (0, 0)),
            pl.BlockSpec((1, E), lambda b, t: (0, 0)),
        ],
        out_specs=pl.BlockSpec(memory_space=pl.ANY),
        out_shape=jax.ShapeDtypeStruct((T, B, E), jnp.float32),
        scratch_shapes=[
            pltpu.VMEM((2, TB, E), jnp.float32),
            pltpu.SemaphoreType.DMA((2,)),
            pltpu.VMEM((2, TB, E), jnp.float32),
            pltpu.SemaphoreType.DMA((2,)),
            pltpu.VMEM((TB, E), jnp.float32),
            pltpu.VMEM((TB, E), jnp.bfloat16),
        ],
        compiler_params=pltpu.CompilerParams(
            dimension_semantics=("arbitrary", "arbitrary"),
            vmem_limit_bytes=100 * 1024 * 1024,
        ),
    )(query, sz, wq_t, bq, rm_bd, wout_t, bout)
    return out


# speedup vs baseline: 1.0542x; 1.0001x over previous
import functools

import numpy as np
import jax
import jax.numpy as jnp
from jax.experimental import pallas as pl
from jax.experimental.pallas import tpu as pltpu

EPS = 1e-8
TB = 1024


def _mm(lhs, rhs):
    return jax.lax.dot_general(
        lhs, rhs, dimension_numbers=(((1,), (0,)), ((), ())),
        preferred_element_type=jnp.float32)


_TWO_OVER_PI = 0.63661977236758134
_PIO2_HI = 1.57079625129699707031
_PIO2_MID = 7.54978941586159635335e-08
_S1, _S2, _S3 = -1.66666546e-01, 8.33216087e-03, -1.95152959e-04
_C1, _C2, _C3, _C4 = (-4.99999997e-01, 4.16666233e-02, -1.38867637e-03,
                      2.43904487e-05)


def _sincos(x):
    ki = jnp.round(x * _TWO_OVER_PI).astype(jnp.int32)
    kf = ki.astype(jnp.float32)
    r = x - kf * _PIO2_HI
    r = r - kf * _PIO2_MID
    r2 = r * r
    sin_r = r + r * r2 * (_S1 + r2 * (_S2 + r2 * _S3))
    cos_r = 1.0 + r2 * (_C1 + r2 * (_C2 + r2 * (_C3 + r2 * _C4)))
    swap = (ki & 1) != 0
    s_base = jnp.where(swap, cos_r, sin_r)
    c_base = jnp.where(swap, sin_r, cos_r)
    s_bits = jax.lax.shift_left((ki & 2), 30)
    c_bits = jax.lax.shift_left(((ki + 1) & 2), 30)
    s_out = jax.lax.bitcast_convert_type(
        jax.lax.bitcast_convert_type(s_base, jnp.int32) ^ s_bits, jnp.float32)
    c_out = jax.lax.bitcast_convert_type(
        jax.lax.bitcast_convert_type(c_base, jnp.int32) ^ c_bits, jnp.float32)
    return s_out, c_out


def _fused_kernel(q_hbm, sz_ref, wq_ref, bq_ref, rm_ref, wout_ref,
                  bout_ref, out_hbm, q_buf, in_sems, out_buf, out_sems,
                  mm_scratch, attn_scratch, *, H, D, P, d_recip, nt, nb):
    b = pl.program_id(0)
    t = pl.program_id(1)
    i = b * nt + t
    slot = jax.lax.rem(i, 2)
    nslot = jax.lax.rem(i + 1, 2)
    t_next = jax.lax.rem(t + 1, nt)
    b_next = b + jnp.where(t + 1 == nt, 1, 0)
    b_next = jnp.minimum(b_next, nb - 1)

    def in_cp(bb, tt, sl):
        return pltpu.make_async_copy(
            q_hbm.at[pl.ds(tt * TB, TB), bb, :],
            q_buf.at[sl], in_sems.at[sl])

    @pl.when(i == 0)
    def _():
        in_cp(b, t, slot).start()

    @pl.when(i + 1 < nb * nt)
    def _():
        in_cp(b_next, t_next, nslot).start()

    in_cp(b, t, slot).wait()

    xq = q_buf[slot].astype(jnp.bfloat16)
    mm_scratch[...] = _mm(xq, wq_ref[...])
    x = (mm_scratch[...] + bq_ref[...]) * d_recip
    xb = x.astype(jnp.bfloat16)
    wx = _mm(xb, rm_ref[...])
    sin_wx, cos_wx = _sincos(wx)
    for h in range(H):
        phi_h = (jnp.concatenate(
            [sin_wx[:, h * P:(h + 1) * P], cos_wx[:, h * P:(h + 1) * P]],
            axis=1) * 0.125).astype(jnp.bfloat16)
        qsz = _mm(phi_h, sz_ref[h])
        qz = jnp.maximum(qsz[:, D:D + 1], EPS)
        attn_scratch[:, h * D:(h + 1) * D] = (
            (qsz[:, :D] / qz).astype(jnp.bfloat16))
    def out_cp(tt, sl):
        return pltpu.make_async_copy(
            out_buf.at[sl], out_hbm.at[pl.ds(tt * TB, TB), b, :],
            out_sems.at[sl])

    @pl.when(i >= 2)
    def _():
        out_cp(t, slot).wait()

    mm_scratch[...] = _mm(attn_scratch[...], wout_ref[...])
    out_buf[slot] = mm_scratch[...] + bout_ref[...]
    out_cp(t, slot).start()

    @pl.when(i == nb * nt - 1)
    def _():
        out_cp(t, slot).wait()

    @pl.when((i == nb * nt - 1) & (nb * nt > 1))
    def _():
        out_cp(t, nslot).wait()


def kernel(query, s, z, random_matrices, W_q, b_q, W_out, b_out):
    T, B, E = query.shape
    _, H, twoP, D = s.shape
    P = twoP // 2
    nt = T // TB

    d_recip = float(np.float32(1.0) / np.float32(float(D) ** 0.25))

    wq_t = W_q.T
    bq = b_q.reshape(1, E)
    wout_t = W_out.T
    bout = b_out.reshape(1, E)

    rm_t = jnp.transpose(random_matrices, (0, 2, 1))
    rm_bd = jax.scipy.linalg.block_diag(*[rm_t[h] for h in range(H)])

    z_col = z[..., None]
    pad = jnp.zeros((B, H, twoP, D - 1), jnp.float32)
    sz = jnp.concatenate([s, z_col, pad], axis=-1)

    grid = (B, nt)
    out = pl.pallas_call(
        functools.partial(_fused_kernel, H=H, D=D, P=P, d_recip=d_recip,
                          nt=nt, nb=B),
        grid=grid,
        in_specs=[
            pl.BlockSpec(memory_space=pl.ANY),
            pl.BlockSpec((None, H, twoP, 2 * D), lambda b, t: (b, 0, 0, 0)),
            pl.BlockSpec((E, E), lambda b, t: (0, 0)),
            pl.BlockSpec((1, E), lambda b, t: (0, 0)),
            pl.BlockSpec((E, H * P), lambda b, t: (0, 0)),
            pl.BlockSpec((E, E), lambda b, t: (0, 0)),
            pl.BlockSpec((1, E), lambda b, t: (0, 0)),
        ],
        out_specs=pl.BlockSpec(memory_space=pl.ANY),
        out_shape=jax.ShapeDtypeStruct((T, B, E), jnp.float32),
        scratch_shapes=[
            pltpu.VMEM((2, TB, E), jnp.float32),
            pltpu.SemaphoreType.DMA((2,)),
            pltpu.VMEM((2, TB, E), jnp.float32),
            pltpu.SemaphoreType.DMA((2,)),
            pltpu.VMEM((TB, E), jnp.float32),
            pltpu.VMEM((TB, E), jnp.bfloat16),
        ],
        compiler_params=pltpu.CompilerParams(
            dimension_semantics=("arbitrary", "arbitrary"),
            vmem_limit_bytes=100 * 1024 * 1024,
        ),
    )(query, sz, wq_t, bq, rm_bd, wout_t, bout)
    return out
